# Initial kernel scaffold; baseline (speedup 1.0000x reference)
import jax
import jax.numpy as jnp
from jax.experimental import pallas as pl


def kernel(x, W_enc, dec_table, b_dec):
    raise NotImplementedError("write your pallas kernel here")


# speedup vs baseline: 1.0017x; 1.0017x over previous
import functools

import jax
import jax.numpy as jnp
from jax import lax
from jax.experimental import pallas as pl

D_IN = 2048
D_HIDDEN = 16384
K = 64
N_TOK = 4096

BT = 2048
BH = 512


def _encode_body(x_ref, w_ref, out_ref):
    x = x_ref[...]
    w = w_ref[...]
    acc = lax.dot_general(
        x, w,
        dimension_numbers=(((1,), (1,)), ((), ())),
        preferred_element_type=jnp.float32,
    )
    out_ref[...] = jnp.maximum(acc, 0.0)


def _encode_pre(x, W_enc):
    grid = (N_TOK // BT, D_HIDDEN // BH)
    return pl.pallas_call(
        _encode_body,
        grid=grid,
        in_specs=[
            pl.BlockSpec((BT, D_IN), lambda i, j: (i, 0)),
            pl.BlockSpec((BH, D_IN), lambda i, j: (j, 0)),
        ],
        out_specs=pl.BlockSpec((BT, BH), lambda i, j: (i, j)),
        out_shape=jax.ShapeDtypeStruct((N_TOK, D_HIDDEN), jnp.float32),
    )(x, W_enc)


def kernel(x, W_enc, dec_table, b_dec):
    xs = x - b_dec
    pre = _encode_pre(xs, W_enc)
    values, indices = jax.lax.top_k(pre, K)
    dec_rows = jnp.take(dec_table, indices, axis=0)
    x_hat = jnp.einsum('ij,ijk->ik', values, dec_rows) + b_dec
    return (x_hat, values, indices)
